# Initial kernel scaffold; baseline (speedup 1.0000x reference)
import jax
import jax.numpy as jnp
from jax.experimental import pallas as pl


def kernel(idx, table, K):
    raise NotImplementedError("write your pallas kernel here")


# speedup vs baseline: 3.9709x; 3.9709x over previous
import functools

import jax
import jax.numpy as jnp
from jax import lax
from jax.experimental import pallas as pl
from jax.experimental.pallas import tpu as pltpu
from jax.experimental.pallas import tpu_sc as plsc

DIM = 64
VOCAB = 100
VOCAB_PAD = 104
BATCH = 16384
HIST = 50
ROWS = BATCH * HIST
NC, NS = 2, 16
NW = NC * NS
CHUNK = 128
CHUNKS = ROWS // CHUNK
CPW = CHUNKS // NW


def _table_body(k_ref, tab_ref, out_ref):
    k = k_ref[0, 0]
    kc = jnp.clip(k, 0.1, 10.0)
    sqrt_k = jnp.sqrt(kc + 1e-08)
    u = tab_ref[:, :]
    norm = jnp.sqrt(jnp.sum(u * u, axis=1, keepdims=True)) + 1e-08
    out_ref[:, :] = jnp.tanh(sqrt_k * norm) * u / (norm + 1e-08)


def _transform_table(table_pad, k_arr):
    return pl.pallas_call(
        _table_body,
        out_shape=jax.ShapeDtypeStruct((VOCAB_PAD, DIM), jnp.float32),
        in_specs=[
            pl.BlockSpec(memory_space=pltpu.SMEM),
            pl.BlockSpec(memory_space=pltpu.VMEM),
        ],
        out_specs=pl.BlockSpec(memory_space=pltpu.VMEM),
    )(k_arr, table_pad)


_MESH = plsc.VectorSubcoreMesh(
    core_axis_name="c", subcore_axis_name="s", num_cores=NC, num_subcores=NS
)


@functools.partial(
    pl.kernel,
    mesh=_MESH,
    compiler_params=pltpu.CompilerParams(use_tc_tiling_on_sc=False),
    out_type=jax.ShapeDtypeStruct((ROWS, DIM), jnp.float32),
    scratch_types=[
        pltpu.VMEM((CPW, CHUNK), jnp.int32),
        pltpu.VMEM((CHUNK, DIM), jnp.float32),
        pltpu.SemaphoreType.DMA,
    ],
)
def _gather(ttable_hbm, idx_hbm, out_hbm, idx_v, rows_v, sem):
    wid = lax.axis_index("s") * NC + lax.axis_index("c")
    c0 = wid * CPW
    pltpu.sync_copy(idx_hbm.at[pl.ds(c0, CPW)], idx_v)

    def body(c, carry):
        pltpu.async_copy(ttable_hbm.at[idx_v.at[c]], rows_v, sem).wait()
        pltpu.sync_copy(rows_v, out_hbm.at[pl.ds((c0 + c) * CHUNK, CHUNK)])
        return carry

    lax.fori_loop(0, CPW, body, 0)


def kernel(idx, table, K):
    table_pad = jnp.pad(table, ((0, VOCAB_PAD - VOCAB), (0, 0)))
    ttable = _transform_table(table_pad, K.reshape(1, 1))
    idx2d = idx.reshape(CHUNKS, CHUNK).astype(jnp.int32)
    out = _gather(ttable, idx2d)
    return out.reshape(BATCH, HIST, DIM)


# speedup vs baseline: 4.1088x; 1.0347x over previous
import functools

import jax
import jax.numpy as jnp
from jax import lax
from jax.experimental import pallas as pl
from jax.experimental.pallas import tpu as pltpu
from jax.experimental.pallas import tpu_sc as plsc

DIM = 64
VOCAB = 100
VOCAB_PAD = 104
BATCH = 16384
HIST = 50
ROWS = BATCH * HIST
NC, NS = 2, 16
NW = NC * NS
CHUNK = 128
CHUNKS = ROWS // CHUNK
CPW = CHUNKS // NW
NBUF = 8


def _table_body(k_ref, tab_ref, out_ref):
    k = k_ref[0, 0]
    kc = jnp.clip(k, 0.1, 10.0)
    sqrt_k = jnp.sqrt(kc + 1e-08)
    u = tab_ref[:, :]
    norm = jnp.sqrt(jnp.sum(u * u, axis=1, keepdims=True)) + 1e-08
    out_ref[:, :] = jnp.tanh(sqrt_k * norm) * u / (norm + 1e-08)


def _transform_table(table_pad, k_arr):
    return pl.pallas_call(
        _table_body,
        out_shape=jax.ShapeDtypeStruct((VOCAB_PAD, DIM), jnp.float32),
        in_specs=[
            pl.BlockSpec(memory_space=pltpu.SMEM),
            pl.BlockSpec(memory_space=pltpu.VMEM),
        ],
        out_specs=pl.BlockSpec(memory_space=pltpu.VMEM),
    )(k_arr, table_pad)


_MESH = plsc.VectorSubcoreMesh(
    core_axis_name="c", subcore_axis_name="s", num_cores=NC, num_subcores=NS
)


@functools.partial(
    pl.kernel,
    mesh=_MESH,
    compiler_params=pltpu.CompilerParams(use_tc_tiling_on_sc=False),
    out_type=jax.ShapeDtypeStruct((ROWS, DIM), jnp.float32),
    scratch_types=[
        pltpu.VMEM((CPW, CHUNK), jnp.int32),
        pltpu.VMEM((NBUF, CHUNK, DIM), jnp.float32),
        pltpu.SemaphoreType.DMA((NBUF,)),
        pltpu.SemaphoreType.DMA((NBUF,)),
    ],
)
def _gather(ttable_hbm, idx_hbm, out_hbm, idx_v, rows_v, gsem, wsem):
    wid = lax.axis_index("s") * NC + lax.axis_index("c")
    c0 = wid * CPW
    pltpu.sync_copy(idx_hbm.at[pl.ds(c0, CPW)], idx_v)

    def body(g, carry):
        base = g * NBUF
        gathers = []
        for b in range(NBUF):
            gathers.append(
                pltpu.async_copy(
                    ttable_hbm.at[idx_v.at[base + b]], rows_v.at[b], gsem.at[b]
                )
            )
        writes = []
        for b in range(NBUF):
            gathers[b].wait()
            writes.append(
                pltpu.async_copy(
                    rows_v.at[b],
                    out_hbm.at[pl.ds((c0 + base + b) * CHUNK, CHUNK)],
                    wsem.at[b],
                )
            )
        for b in range(NBUF):
            writes[b].wait()
        return carry

    lax.fori_loop(0, CPW // NBUF, body, 0)


def kernel(idx, table, K):
    table_pad = jnp.pad(table, ((0, VOCAB_PAD - VOCAB), (0, 0)))
    ttable = _transform_table(table_pad, K.reshape(1, 1))
    idx2d = idx.reshape(CHUNKS, CHUNK).astype(jnp.int32)
    out = _gather(ttable, idx2d)
    return out.reshape(BATCH, HIST, DIM)


# speedup vs baseline: 6.1006x; 1.4848x over previous
import functools

import jax
import jax.numpy as jnp
from jax import lax
from jax.experimental import pallas as pl
from jax.experimental.pallas import tpu as pltpu
from jax.experimental.pallas import tpu_sc as plsc

DIM = 64
VOCAB = 100
VOCAB_PAD = 104
BATCH = 16384
HIST = 50
ROWS = BATCH * HIST
NC, NS = 2, 16
NW = NC * NS
CHUNK = 128
CHUNKS = ROWS // CHUNK
CPW = CHUNKS // NW
NBUF = 8


def _table_body(k_ref, tab_ref, out_ref):
    k = k_ref[0, 0]
    kc = jnp.clip(k, 0.1, 10.0)
    sqrt_k = jnp.sqrt(kc + 1e-08)
    u = tab_ref[:, :]
    norm = jnp.sqrt(jnp.sum(u * u, axis=1, keepdims=True)) + 1e-08
    out_ref[:, :] = jnp.tanh(sqrt_k * norm) * u / (norm + 1e-08)


def _transform_table(table_pad, k_arr):
    return pl.pallas_call(
        _table_body,
        out_shape=jax.ShapeDtypeStruct((VOCAB_PAD, DIM), jnp.float32),
        in_specs=[
            pl.BlockSpec(memory_space=pltpu.SMEM),
            pl.BlockSpec(memory_space=pltpu.VMEM),
        ],
        out_specs=pl.BlockSpec(memory_space=pltpu.VMEM),
    )(k_arr, table_pad)


_MESH = plsc.VectorSubcoreMesh(
    core_axis_name="c", subcore_axis_name="s", num_cores=NC, num_subcores=NS
)


@functools.partial(
    pl.kernel,
    mesh=_MESH,
    compiler_params=pltpu.CompilerParams(use_tc_tiling_on_sc=False),
    out_type=jax.ShapeDtypeStruct((ROWS, DIM), jnp.float32),
    scratch_types=[
        pltpu.VMEM((CPW, CHUNK), jnp.int32),
        pltpu.VMEM((NBUF, CHUNK, DIM), jnp.float32),
        pltpu.SemaphoreType.DMA((NBUF,)),
        pltpu.SemaphoreType.DMA((NBUF,)),
    ],
)
def _gather(ttable_hbm, idx_hbm, out_hbm, idx_v, rows_v, gsem, wsem):
    wid = lax.axis_index("s") * NC + lax.axis_index("c")
    c0 = wid * CPW
    pltpu.sync_copy(idx_hbm.at[pl.ds(c0, CPW)], idx_v)

    def body(g, carry):
        base = g * NBUF
        gathers = []
        for b in range(NBUF):
            gathers.append(
                pltpu.async_copy(
                    ttable_hbm.at[idx_v.at[base + b]], rows_v.at[b], gsem.at[b]
                )
            )
        writes = []
        for b in range(NBUF):
            gathers[b].wait()
            writes.append(
                pltpu.async_copy(
                    rows_v.at[b],
                    out_hbm.at[pl.ds((c0 + base + b) * CHUNK, CHUNK)],
                    wsem.at[b],
                )
            )
        for b in range(NBUF):
            writes[b].wait()
        return carry

    lax.fori_loop(0, CPW // NBUF, body, 0)


def kernel(idx, table, K):
    table_pad = jnp.pad(table, ((0, VOCAB_PAD - VOCAB), (0, 0)))
    ttable = _transform_table(table_pad, K.reshape(1, 1))
    ttable_rep = jnp.tile(ttable, (NW, 1)).reshape(NW * VOCAB_PAD, DIM)
    idx2d = idx.reshape(CHUNKS, CHUNK).astype(jnp.int32)
    bias = (jnp.arange(CHUNKS, dtype=jnp.int32)[:, None] // CPW) * VOCAB_PAD
    out = _gather(ttable_rep, idx2d + bias)
    return out.reshape(BATCH, HIST, DIM)


# speedup vs baseline: 6.1507x; 1.0082x over previous
import functools

import jax
import jax.numpy as jnp
from jax import lax
from jax.experimental import pallas as pl
from jax.experimental.pallas import tpu as pltpu
from jax.experimental.pallas import tpu_sc as plsc

DIM = 64
D128 = 128
VOCAB = 100
VOCAB_PAD = 104
BATCH = 16384
HIST = 50
SLOT = 64
NC, NS = 2, 16
NW = NC * NS
BPW = BATCH // NW
IDX_ROWS = BATCH // 2
IDX_ROWS_W = BPW // 2
NBUF = 4


def _table_body(k_ref, tab_ref, out_ref):
    k = k_ref[0, 0]
    kc = jnp.clip(k, 0.1, 10.0)
    sqrt_k = jnp.sqrt(kc + 1e-08)
    u = tab_ref[:, :]
    norm = jnp.sqrt(jnp.sum(u * u, axis=1, keepdims=True)) + 1e-08
    res = jnp.tanh(sqrt_k * norm) * u / (norm + 1e-08)
    out_ref[:, :] = jnp.pad(res, ((0, 0), (0, D128 - DIM)))


def _transform_table(table_pad, k_arr):
    return pl.pallas_call(
        _table_body,
        grid=(NW,),
        out_shape=jax.ShapeDtypeStruct((NW * VOCAB_PAD, D128), jnp.float32),
        in_specs=[
            pl.BlockSpec(memory_space=pltpu.SMEM),
            pl.BlockSpec((VOCAB_PAD, DIM), lambda i: (0, 0)),
        ],
        out_specs=pl.BlockSpec((VOCAB_PAD, D128), lambda i: (i, 0)),
    )(k_arr, table_pad)


_MESH = plsc.VectorSubcoreMesh(
    core_axis_name="c", subcore_axis_name="s", num_cores=NC, num_subcores=NS
)


@functools.partial(
    pl.kernel,
    mesh=_MESH,
    out_type=jax.ShapeDtypeStruct((BATCH, HIST, DIM), jnp.float32),
    scratch_types=[
        pltpu.VMEM((IDX_ROWS_W, D128), jnp.int32),
        pltpu.VMEM((NBUF, HIST, D128), jnp.float32),
        pltpu.VMEM((NBUF, HIST, DIM), jnp.float32),
        pltpu.SemaphoreType.DMA((NBUF,)),
        pltpu.SemaphoreType.DMA((NBUF,)),
    ],
)
def _gather(ttable_hbm, idx_hbm, out_hbm, idx_v, g_v, rows_v, gsem, wsem):
    wid = lax.axis_index("s") * NC + lax.axis_index("c")
    e0 = wid * BPW
    pltpu.sync_copy(idx_hbm.at[pl.ds(wid * IDX_ROWS_W, IDX_ROWS_W)], idx_v)

    def body(g, carry):
        base = g * NBUF
        gathers = []
        for b in range(NBUF):
            r = (base + b) // 2
            gathers.append(
                pltpu.async_copy(
                    ttable_hbm.at[idx_v.at[r, pl.ds(SLOT * (b % 2), HIST)]],
                    g_v.at[b],
                    gsem.at[b],
                )
            )
        writes = []
        for b in range(NBUF):
            gathers[b].wait()

            def copy_row(j, c):
                for c4 in range(4):
                    rows_v[b, j, pl.ds(16 * c4, 16)] = g_v[b, j, pl.ds(16 * c4, 16)]
                return c

            lax.fori_loop(0, HIST, copy_row, 0)
            writes.append(
                pltpu.async_copy(
                    rows_v.at[b],
                    out_hbm.at[e0 + base + b],
                    wsem.at[b],
                )
            )
        for b in range(NBUF):
            writes[b].wait()
        return carry

    lax.fori_loop(0, BPW // NBUF, body, 0)


def kernel(idx, table, K):
    table_pad = jnp.pad(table, ((0, VOCAB_PAD - VOCAB), (0, 0)))
    ttable = _transform_table(table_pad, K.reshape(1, 1))
    bias = (jnp.arange(BATCH, dtype=jnp.int32)[:, None] // BPW) * VOCAB_PAD
    idxp = jnp.pad(idx.astype(jnp.int32) + bias, ((0, 0), (0, SLOT - HIST)))
    return _gather(ttable, idxp.reshape(IDX_ROWS, D128))
